# baseline (speedup 1.0000x reference)
import functools

import jax
import jax.numpy as jnp
from jax import lax
from jax.experimental import pallas as pl
from jax.experimental.pallas import tpu as pltpu
from jax.experimental.pallas import tpu_sc as plsc

B = 4096
L = 200
EMB = 64
NCLS = 100
VOCAB = 1000000

NC, NS = 2, 16
NW = NC * NS
RPW = B // NW
C0 = 128
C1 = L - C0
NQ = EMB // 16
PAIR = 2 * EMB
HALF = RPW // 2


def _sc_pool_sum(xk, xo, tab2):
    mesh = plsc.VectorSubcoreMesh(core_axis_name="c", subcore_axis_name="s")

    @functools.partial(
        pl.kernel,
        out_type=jax.ShapeDtypeStruct((B, EMB), jnp.float32),
        mesh=mesh,
        scratch_types=[
            pltpu.VMEM((HALF, L), jnp.int32),
            pltpu.VMEM((HALF, L), jnp.int32),
            pltpu.VMEM((L, PAIR), jnp.float32),
            pltpu.VMEM((L, PAIR), jnp.float32),
            pltpu.VMEM((RPW, EMB), jnp.float32),
            pltpu.SemaphoreType.DMA,
            pltpu.SemaphoreType.DMA,
        ],
    )
    def pool_kernel(xk_hbm, xo_hbm, tab_hbm, out_hbm,
                    kidx_v, poff_v, rows0, rows1, acc_v, sem0, sem1):
        wid = lax.axis_index("s") * NC + lax.axis_index("c")
        base = wid * RPW

        def issue(r, rows_v, sem):
            pltpu.async_copy(
                tab_hbm.at[kidx_v.at[r, pl.ds(0, C0)]], rows_v.at[pl.ds(0, C0), :], sem)
            pltpu.async_copy(
                tab_hbm.at[kidx_v.at[r, pl.ds(C0, C1)]], rows_v.at[pl.ds(C0, C1), :], sem)

        def drain(rows_v, sem):
            pltpu.make_async_copy(tab_hbm.at[pl.ds(0, L), :], rows_v, sem).wait()

        for g in range(RPW // HALF):
            gbase = g * HALF
            pltpu.sync_copy(xk_hbm.at[pl.ds(base + gbase, HALF), :], kidx_v)
            pltpu.sync_copy(xo_hbm.at[pl.ds(base + gbase, HALF), :], poff_v)

            def accum(r, rows_v, gbase=gbase):
                zero = jnp.zeros((16,), jnp.float32)

                def step(j_base, offv, l, accs):
                    off = pl.multiple_of(offv[l], EMB)
                    return tuple(a + rows_v[j_base + l, pl.ds(off + 16 * q, 16)]
                                 for q, a in enumerate(accs))

                def body(jj, accs):
                    j_base = 16 * jj
                    offv = poff_v[r, pl.ds(j_base, 16)]
                    for l in range(16):
                        accs = step(j_base, offv, l, accs)
                    return accs

                accs = lax.fori_loop(0, L // 16, body, (zero,) * NQ)
                offv = poff_v[r, pl.ds(L - 16, 16)]
                for l in range(16 - (L - 16 * (L // 16)), 16):
                    accs = step(L - 16, offv, l, accs)
                for q in range(NQ):
                    acc_v[gbase + r, pl.ds(16 * q, 16)] = accs[q]

            issue(0, rows0, sem0)

            def outer(t, carry, accum=accum):
                r = 2 * t
                issue(r + 1, rows1, sem1)
                drain(rows0, sem0)
                accum(r, rows0)

                @pl.when(r + 2 < HALF)
                def _():
                    issue(r + 2, rows0, sem0)

                drain(rows1, sem1)
                accum(r + 1, rows1)
                return carry

            lax.fori_loop(0, HALF // 2, outer, 0)

        pltpu.sync_copy(acc_v, out_hbm.at[pl.ds(base, RPW), :])

    return pool_kernel(xk, xo, tab2)


def _tc_head(pooled_sum, Wt, b2):
    def head_kernel(p_ref, w_ref, b_ref, o_ref):
        o_ref[...] = (
            jnp.dot(p_ref[...], w_ref[...], preferred_element_type=jnp.float32)
            * (1.0 / L)
            + b_ref[...]
        )

    return pl.pallas_call(
        head_kernel,
        out_shape=jax.ShapeDtypeStruct((B, NCLS), jnp.float32),
    )(pooled_sum, Wt, b2)


def kernel(x, emb_table, W, b):
    x = x.astype(jnp.int32)
    xk = x >> 1
    xo = (x & 1) << 6
    tab2 = emb_table.reshape(VOCAB // 2, PAIR)
    pooled_sum = _sc_pool_sum(xk, xo, tab2)
    return _tc_head(pooled_sum, W.T, b.reshape(1, NCLS))


# speedup vs baseline: 1.6768x; 1.6768x over previous
import functools

import jax
import jax.numpy as jnp
from jax import lax
from jax.experimental import pallas as pl
from jax.experimental.pallas import tpu as pltpu
from jax.experimental.pallas import tpu_sc as plsc

B = 4096
L = 200
EMB = 64
NCLS = 100
VOCAB = 1000000

NC, NS = 2, 16
NW = NC * NS
RPW = B // NW
C0 = 128
C1 = L - C0
NQ = EMB // 16

VBH = 2048
NBLK = -(-VOCAB // (2 * VBH))
VMAIN = (NBLK - 1) * 2 * VBH
VPAD = NBLK * 2 * VBH
LASTB = 2 * (NBLK - 1)


def _tc_pack(tabT):
    def pack_kernel(a_ref, b_ref, o_ref):
        o_ref[:, 0:EMB] = a_ref[...].T
        o_ref[:, EMB:2 * EMB] = b_ref[...].T

    return pl.pallas_call(
        pack_kernel,
        grid=(NBLK,),
        in_specs=[
            pl.BlockSpec((EMB, VBH), lambda i: (0, jnp.minimum(2 * i, LASTB))),
            pl.BlockSpec((EMB, VBH), lambda i: (0, jnp.minimum(2 * i + 1, LASTB))),
        ],
        out_specs=pl.BlockSpec((VBH, 2 * EMB), lambda i: (i, 0)),
        out_shape=jax.ShapeDtypeStruct((VPAD // 2, 2 * EMB), jnp.float32),
    )(tabT, tabT)


def _sc_pool_sum(x, tab):
    mesh = plsc.VectorSubcoreMesh(core_axis_name="c", subcore_axis_name="s")

    @functools.partial(
        pl.kernel,
        out_type=jax.ShapeDtypeStruct((B, EMB), jnp.float32),
        mesh=mesh,
        compiler_params=pltpu.CompilerParams(use_tc_tiling_on_sc=False),
        scratch_types=[
            pltpu.VMEM((RPW, L), jnp.int32),
            pltpu.VMEM((L, EMB), jnp.float32),
            pltpu.VMEM((L, EMB), jnp.float32),
            pltpu.VMEM((RPW, EMB), jnp.float32),
            pltpu.SemaphoreType.DMA,
            pltpu.SemaphoreType.DMA,
        ],
    )
    def pool_kernel(x_hbm, tab_hbm, out_hbm, idx_v, rows0, rows1, acc_v, sem0, sem1):
        wid = lax.axis_index("s") * NC + lax.axis_index("c")
        base = wid * RPW
        pltpu.sync_copy(x_hbm.at[pl.ds(base, RPW), :], idx_v)

        def issue(r, rows_v, sem):
            pltpu.async_copy(
                tab_hbm.at[idx_v.at[r, pl.ds(0, C0)]], rows_v.at[pl.ds(0, C0), :], sem)
            pltpu.async_copy(
                tab_hbm.at[idx_v.at[r, pl.ds(C0, C1)]], rows_v.at[pl.ds(C0, C1), :], sem)

        def drain(rows_v, sem):
            pltpu.make_async_copy(tab_hbm.at[pl.ds(0, L), :], rows_v, sem).wait()

        def accum(r, rows_v):
            zero = jnp.zeros((16,), jnp.float32)

            def body(j, accs):
                return tuple(a + rows_v[j, pl.ds(16 * q, 16)]
                             for q, a in enumerate(accs))

            accs = lax.fori_loop(0, L, body, (zero,) * NQ)
            for q in range(NQ):
                acc_v[r, pl.ds(16 * q, 16)] = accs[q]

        issue(0, rows0, sem0)

        def outer(t, carry):
            r = 2 * t
            issue(r + 1, rows1, sem1)
            drain(rows0, sem0)
            accum(r, rows0)

            @pl.when(r + 2 < RPW)
            def _():
                issue(r + 2, rows0, sem0)

            drain(rows1, sem1)
            accum(r + 1, rows1)
            return carry

        lax.fori_loop(0, RPW // 2, outer, 0)
        pltpu.sync_copy(acc_v, out_hbm.at[pl.ds(base, RPW), :])

    return pool_kernel(x, tab)


def _tc_head(pooled_sum, Wt, b2):
    def head_kernel(p_ref, w_ref, b_ref, o_ref):
        o_ref[...] = (
            jnp.dot(p_ref[...], w_ref[...], preferred_element_type=jnp.float32)
            * (1.0 / L)
            + b_ref[...]
        )

    return pl.pallas_call(
        head_kernel,
        out_shape=jax.ShapeDtypeStruct((B, NCLS), jnp.float32),
    )(pooled_sum, Wt, b2)


def kernel(x, emb_table, W, b):
    x = x.astype(jnp.int32)
    xr = jnp.where(x < VMAIN,
                   (x & ~4095) + 2 * (x & 2047) + ((x >> 11) & 1),
                   2 * x - VMAIN)
    tab2 = _tc_pack(emb_table.T)
    tab = tab2.reshape(VPAD, EMB)
    pooled_sum = _sc_pool_sum(xr, tab)
    return _tc_head(pooled_sum, W.T, b.reshape(1, NCLS))


# speedup vs baseline: 2.0992x; 1.2519x over previous
import functools

import jax
import jax.numpy as jnp
from jax import lax
from jax.experimental import pallas as pl
from jax.experimental.pallas import tpu as pltpu
from jax.experimental.pallas import tpu_sc as plsc

B = 4096
L = 200
EMB = 64
NCLS = 100
VOCAB = 1000000

NC, NS = 2, 16
NW = NC * NS
RPW = B // NW
C0 = 128
C1 = L - C0
NQ = EMB // 16

VBH = 8192
NBLK = -(-VOCAB // (2 * VBH))
VMAIN = (NBLK - 1) * 2 * VBH
VPAD = NBLK * 2 * VBH
LASTB = 2 * (NBLK - 1)


def _tc_pack(tabT):
    def pack_kernel(a_ref, b_ref, o_ref):
        o_ref[:, 0:EMB] = a_ref[...].T
        o_ref[:, EMB:2 * EMB] = b_ref[...].T

    return pl.pallas_call(
        pack_kernel,
        grid=(NBLK,),
        in_specs=[
            pl.BlockSpec((EMB, VBH), lambda i: (0, jnp.minimum(2 * i, LASTB))),
            pl.BlockSpec((EMB, VBH), lambda i: (0, jnp.minimum(2 * i + 1, LASTB))),
        ],
        out_specs=pl.BlockSpec((VBH, 2 * EMB), lambda i: (i, 0)),
        out_shape=jax.ShapeDtypeStruct((VPAD // 2, 2 * EMB), jnp.float32),
    )(tabT, tabT)


def _sc_pool_sum(x, tab):
    mesh = plsc.VectorSubcoreMesh(core_axis_name="c", subcore_axis_name="s")

    @functools.partial(
        pl.kernel,
        out_type=jax.ShapeDtypeStruct((B, EMB), jnp.float32),
        mesh=mesh,
        compiler_params=pltpu.CompilerParams(use_tc_tiling_on_sc=False),
        scratch_types=[
            pltpu.VMEM((RPW, L), jnp.int32),
            pltpu.VMEM((L, EMB), jnp.float32),
            pltpu.VMEM((L, EMB), jnp.float32),
            pltpu.VMEM((RPW, EMB), jnp.float32),
            pltpu.SemaphoreType.DMA,
            pltpu.SemaphoreType.DMA,
        ],
    )
    def pool_kernel(x_hbm, tab_hbm, out_hbm, idx_v, rows0, rows1, acc_v, sem0, sem1):
        wid = lax.axis_index("s") * NC + lax.axis_index("c")
        base = wid * RPW
        pltpu.sync_copy(x_hbm.at[pl.ds(base, RPW), :], idx_v)

        def issue(r, rows_v, sem):
            pltpu.async_copy(
                tab_hbm.at[idx_v.at[r, pl.ds(0, C0)]], rows_v.at[pl.ds(0, C0), :], sem)
            pltpu.async_copy(
                tab_hbm.at[idx_v.at[r, pl.ds(C0, C1)]], rows_v.at[pl.ds(C0, C1), :], sem)

        def drain(rows_v, sem):
            pltpu.make_async_copy(tab_hbm.at[pl.ds(0, L), :], rows_v, sem).wait()

        def accum(r, rows_v):
            zero = jnp.zeros((16,), jnp.float32)

            def body(j, accs):
                return tuple(a + rows_v[j, pl.ds(16 * q, 16)]
                             for q, a in enumerate(accs))

            accs = lax.fori_loop(0, L, body, (zero,) * NQ)
            for q in range(NQ):
                acc_v[r, pl.ds(16 * q, 16)] = accs[q]

        issue(0, rows0, sem0)

        def outer(t, carry):
            r = 2 * t
            issue(r + 1, rows1, sem1)
            drain(rows0, sem0)
            accum(r, rows0)

            @pl.when(r + 2 < RPW)
            def _():
                issue(r + 2, rows0, sem0)

            drain(rows1, sem1)
            accum(r + 1, rows1)
            return carry

        lax.fori_loop(0, RPW // 2, outer, 0)
        pltpu.sync_copy(acc_v, out_hbm.at[pl.ds(base, RPW), :])

    return pool_kernel(x, tab)


def _tc_head(pooled_sum, Wt, b2):
    def head_kernel(p_ref, w_ref, b_ref, o_ref):
        o_ref[...] = (
            jnp.dot(p_ref[...], w_ref[...], preferred_element_type=jnp.float32)
            * (1.0 / L)
            + b_ref[...]
        )

    return pl.pallas_call(
        head_kernel,
        out_shape=jax.ShapeDtypeStruct((B, NCLS), jnp.float32),
    )(pooled_sum, Wt, b2)


def kernel(x, emb_table, W, b):
    x = x.astype(jnp.int32)
    xr = jnp.where(x < VMAIN,
                   (x & ~(2 * VBH - 1)) + 2 * (x & (VBH - 1)) + ((x // VBH) & 1),
                   2 * x - VMAIN)
    tab2 = _tc_pack(emb_table.T)
    tab = tab2.reshape(VPAD, EMB)
    pooled_sum = _sc_pool_sum(xr, tab)
    return _tc_head(pooled_sum, W.T, b.reshape(1, NCLS))


# speedup vs baseline: 2.1912x; 1.0438x over previous
import functools

import jax
import jax.numpy as jnp
from jax import lax
from jax.experimental import pallas as pl
from jax.experimental.pallas import tpu as pltpu
from jax.experimental.pallas import tpu_sc as plsc

B = 4096
L = 200
EMB = 64
NCLS = 100
VOCAB = 1000000

NC, NS = 2, 16
NW = NC * NS
RPW = B // NW
C0 = 128
C1 = L - C0
NQ = EMB // 16

VBH = 16384
NBLK = -(-VOCAB // (2 * VBH))
VPAD = NBLK * 2 * VBH
LASTB = (VOCAB - 1) // VBH


def _tc_pack(tabT):
    def pack_kernel(a_ref, b_ref, o_ref):
        o_ref[:, 0:EMB] = a_ref[...].T
        o_ref[:, EMB:2 * EMB] = b_ref[...].T

    return pl.pallas_call(
        pack_kernel,
        grid=(NBLK,),
        in_specs=[
            pl.BlockSpec((EMB, VBH), lambda i: (0, jnp.minimum(2 * i, LASTB))),
            pl.BlockSpec((EMB, VBH), lambda i: (0, jnp.minimum(2 * i + 1, LASTB))),
        ],
        out_specs=pl.BlockSpec((VBH, 2 * EMB), lambda i: (i, 0)),
        out_shape=jax.ShapeDtypeStruct((VPAD // 2, 2 * EMB), jnp.float32),
    )(tabT, tabT)


def _sc_pool_sum(x, tab):
    mesh = plsc.VectorSubcoreMesh(core_axis_name="c", subcore_axis_name="s")

    @functools.partial(
        pl.kernel,
        out_type=jax.ShapeDtypeStruct((B, EMB), jnp.float32),
        mesh=mesh,
        compiler_params=pltpu.CompilerParams(use_tc_tiling_on_sc=False),
        scratch_types=[
            pltpu.VMEM((RPW, L), jnp.int32),
            pltpu.VMEM((L, EMB), jnp.float32),
            pltpu.VMEM((L, EMB), jnp.float32),
            pltpu.VMEM((RPW, EMB), jnp.float32),
            pltpu.SemaphoreType.DMA,
            pltpu.SemaphoreType.DMA,
        ],
    )
    def pool_kernel(x_hbm, tab_hbm, out_hbm, idx_v, rows0, rows1, acc_v, sem0, sem1):
        wid = lax.axis_index("s") * NC + lax.axis_index("c")
        base = wid * RPW
        pltpu.sync_copy(x_hbm.at[pl.ds(base, RPW), :], idx_v)

        def issue(r, rows_v, sem):
            pltpu.async_copy(
                tab_hbm.at[idx_v.at[r, pl.ds(0, C0)]], rows_v.at[pl.ds(0, C0), :], sem)
            pltpu.async_copy(
                tab_hbm.at[idx_v.at[r, pl.ds(C0, C1)]], rows_v.at[pl.ds(C0, C1), :], sem)

        def drain(rows_v, sem):
            pltpu.make_async_copy(tab_hbm.at[pl.ds(0, L), :], rows_v, sem).wait()

        def accum(r, rows_v):
            zero = jnp.zeros((16,), jnp.float32)

            def body(j, accs):
                return tuple(a + rows_v[j, pl.ds(16 * q, 16)]
                             for q, a in enumerate(accs))

            accs = lax.fori_loop(0, L, body, (zero,) * NQ)
            for q in range(NQ):
                acc_v[r, pl.ds(16 * q, 16)] = accs[q]

        issue(0, rows0, sem0)

        def outer(t, carry):
            r = 2 * t
            issue(r + 1, rows1, sem1)
            drain(rows0, sem0)
            accum(r, rows0)

            @pl.when(r + 2 < RPW)
            def _():
                issue(r + 2, rows0, sem0)

            drain(rows1, sem1)
            accum(r + 1, rows1)
            return carry

        lax.fori_loop(0, RPW // 2, outer, 0)
        pltpu.sync_copy(acc_v, out_hbm.at[pl.ds(base, RPW), :])

    return pool_kernel(x, tab)


def _tc_head(pooled_sum, Wt, b2):
    def head_kernel(p_ref, w_ref, b_ref, o_ref):
        o_ref[...] = (
            jnp.dot(p_ref[...], w_ref[...], preferred_element_type=jnp.float32)
            * (1.0 / L)
            + b_ref[...]
        )

    return pl.pallas_call(
        head_kernel,
        out_shape=jax.ShapeDtypeStruct((B, NCLS), jnp.float32),
    )(pooled_sum, Wt, b2)


def kernel(x, emb_table, W, b):
    x = x.astype(jnp.int32)
    xr = (x & ~(2 * VBH - 1)) + 2 * (x & (VBH - 1)) + ((x // VBH) & 1)
    tab2 = _tc_pack(emb_table.T)
    tab = tab2.reshape(VPAD, EMB)
    pooled_sum = _sc_pool_sum(xr, tab)
    return _tc_head(pooled_sum, W.T, b.reshape(1, NCLS))
